# baseline (speedup 1.0000x reference)
import functools

import jax
import jax.numpy as jnp
from jax import lax
from jax.experimental import pallas as pl
from jax.experimental.pallas import tpu as pltpu

B, DIM, T = 16, 768, 576
K, EDIM = 8192, 256
EPS = 1e-12
KT = 1024
BB = 4
TT = BB * T

_CONTRACT_LAST = (((1,), (1,)), ((), ()))


def _nsvq_kernel(x_ref, cb_ref, w_in_ref, b_in_ref, w_out_ref, b_out_ref,
                 rand_ref, out_ref, cn_ref, cb_bf_ref):
    @pl.when(pl.program_id(0) == 0)
    def _init_codebook():
        cb32 = cb_ref[...]
        cb_bf_ref[...] = cb32.astype(jnp.bfloat16)
        ones = jnp.ones((1, EDIM), dtype=jnp.float32)
        cn_ref[...] = lax.dot_general(ones, cb32 * cb32, _CONTRACT_LAST,
                                      preferred_element_type=jnp.float32)

    x = x_ref[...].reshape(TT, DIM)
    enc = lax.dot_general(x, w_in_ref[...], _CONTRACT_LAST,
                          preferred_element_type=jnp.float32)
    enc = enc + b_in_ref[...]
    ennorm2 = jnp.sum(enc * enc, axis=1, keepdims=True)

    enc_b = (-2.0 * enc).astype(jnp.bfloat16)
    m = jnp.full((TT, KT), jnp.inf, dtype=jnp.float32)
    for kt in range(K // KT):
        cb = cb_bf_ref[pl.ds(kt * KT, KT), :]
        s = lax.dot_general(enc_b, cb, _CONTRACT_LAST,
                            preferred_element_type=jnp.float32)
        m = jnp.minimum(m, s + cn_ref[:, pl.ds(kt * KT, KT)])
    mmin = jnp.min(m, axis=1, keepdims=True)

    r = rand_ref[...].reshape(TT, EDIM)
    rnorm = jnp.sqrt(jnp.sum(r * r, axis=1, keepdims=True))
    res = jnp.sqrt(jnp.maximum(ennorm2 + mmin, 0.0))
    scale = res / (rnorm + EPS)
    q = enc + scale * r
    out = lax.dot_general(q, w_out_ref[...], _CONTRACT_LAST,
                          preferred_element_type=jnp.float32)
    out_ref[...] = (out + b_out_ref[...]).reshape(BB, T, DIM)


@functools.partial(jax.jit, static_argnames=())
def kernel(input_data, codebooks, W_in, b_in, W_out, b_out, random_vector):
    xt = jnp.transpose(input_data, (0, 2, 1))
    rand3 = random_vector.reshape(B, T, EDIM)
    b_in2 = b_in.reshape(1, EDIM)
    b_out2 = b_out.reshape(1, DIM)

    out = pl.pallas_call(
        _nsvq_kernel,
        grid=(B // BB,),
        in_specs=[
            pl.BlockSpec((BB, T, DIM), lambda b: (b, 0, 0)),
            pl.BlockSpec((K, EDIM), lambda b: (0, 0)),
            pl.BlockSpec((EDIM, DIM), lambda b: (0, 0)),
            pl.BlockSpec((1, EDIM), lambda b: (0, 0)),
            pl.BlockSpec((DIM, EDIM), lambda b: (0, 0)),
            pl.BlockSpec((1, DIM), lambda b: (0, 0)),
            pl.BlockSpec((BB, T, EDIM), lambda b: (b, 0, 0)),
        ],
        out_specs=pl.BlockSpec((BB, T, DIM), lambda b: (b, 0, 0)),
        out_shape=jax.ShapeDtypeStruct((B, T, DIM), jnp.float32),
        scratch_shapes=[pltpu.VMEM((1, K), jnp.float32),
                        pltpu.VMEM((K, EDIM), jnp.bfloat16)],
    )(xt, codebooks, W_in, b_in2, W_out, b_out2, rand3)
    return jnp.transpose(out, (0, 2, 1))


# speedup vs baseline: 1.0228x; 1.0228x over previous
import functools

import jax
import jax.numpy as jnp
from jax import lax
from jax.experimental import pallas as pl
from jax.experimental.pallas import tpu as pltpu

B, DIM, T = 16, 768, 576
K, EDIM = 8192, 256
EPS = 1e-12
KT = 1024
BB = 2
TT = BB * T

_CONTRACT_LAST = (((1,), (1,)), ((), ()))


def _nsvq_kernel(x_ref, cb_ref, w_in_ref, b_in_ref, w_out_ref, b_out_ref,
                 rand_ref, out_ref, cn_ref, cb_bf_ref):
    @pl.when(pl.program_id(0) == 0)
    def _init_codebook():
        cb32 = cb_ref[...]
        cb_bf_ref[...] = cb32.astype(jnp.bfloat16)
        ones = jnp.ones((1, EDIM), dtype=jnp.float32)
        cn_ref[...] = lax.dot_general(ones, cb32 * cb32, _CONTRACT_LAST,
                                      preferred_element_type=jnp.float32)

    x = x_ref[...].reshape(TT, DIM)
    enc = lax.dot_general(x, w_in_ref[...], _CONTRACT_LAST,
                          preferred_element_type=jnp.float32)
    enc = enc + b_in_ref[...]
    ennorm2 = jnp.sum(enc * enc, axis=1, keepdims=True)

    enc_b = (-2.0 * enc).astype(jnp.bfloat16)
    m = None
    for kt in range(K // KT):
        cb = cb_bf_ref[pl.ds(kt * KT, KT), :]
        s = lax.dot_general(enc_b, cb, _CONTRACT_LAST,
                            preferred_element_type=jnp.float32)
        d = s + cn_ref[:, pl.ds(kt * KT, KT)]
        m = d if m is None else jnp.minimum(m, d)
    mmin = jnp.min(m, axis=1, keepdims=True)

    r = rand_ref[...].reshape(TT, EDIM)
    rnorm = jnp.sqrt(jnp.sum(r * r, axis=1, keepdims=True))
    res = jnp.sqrt(jnp.maximum(ennorm2 + mmin, 0.0))
    scale = res / (rnorm + EPS)
    q = enc + scale * r
    out = lax.dot_general(q, w_out_ref[...], _CONTRACT_LAST,
                          preferred_element_type=jnp.float32)
    out_ref[...] = (out + b_out_ref[...]).reshape(BB, T, DIM)


@functools.partial(jax.jit, static_argnames=())
def kernel(input_data, codebooks, W_in, b_in, W_out, b_out, random_vector):
    xt = jnp.transpose(input_data, (0, 2, 1))
    rand3 = random_vector.reshape(B, T, EDIM)
    b_in2 = b_in.reshape(1, EDIM)
    b_out2 = b_out.reshape(1, DIM)

    out = pl.pallas_call(
        _nsvq_kernel,
        grid=(B // BB,),
        in_specs=[
            pl.BlockSpec((BB, T, DIM), lambda b: (b, 0, 0)),
            pl.BlockSpec((K, EDIM), lambda b: (0, 0)),
            pl.BlockSpec((EDIM, DIM), lambda b: (0, 0)),
            pl.BlockSpec((1, EDIM), lambda b: (0, 0)),
            pl.BlockSpec((DIM, EDIM), lambda b: (0, 0)),
            pl.BlockSpec((1, DIM), lambda b: (0, 0)),
            pl.BlockSpec((BB, T, EDIM), lambda b: (b, 0, 0)),
        ],
        out_specs=pl.BlockSpec((BB, T, DIM), lambda b: (b, 0, 0)),
        out_shape=jax.ShapeDtypeStruct((B, T, DIM), jnp.float32),
        scratch_shapes=[pltpu.VMEM((1, K), jnp.float32),
                        pltpu.VMEM((K, EDIM), jnp.bfloat16)],
    )(xt, codebooks, W_in, b_in2, W_out, b_out2, rand3)
    return jnp.transpose(out, (0, 2, 1))
